# baseline (speedup 1.0000x reference)
import functools
import math

import jax
import jax.numpy as jnp
from jax.experimental import pallas as pl
from jax.experimental.pallas import tpu as pltpu

VTV_NF = 64


def _silu(z):
    return z * jax.nn.sigmoid(z)


def _edge_mlp_body(hr, hc, w1a, w1b, b1, w2, b2, out):
    z = jnp.dot(hr[...], w1a[...], preferred_element_type=jnp.float32)
    z += jnp.dot(hc[...], w1b[...], preferred_element_type=jnp.float32)
    z = _silu(z + b1[...])
    out[...] = jnp.dot(z, w2[...], preferred_element_type=jnp.float32) + b2[...]


def _edge_mlp(hr, hc, We1, be1, We2, be2, block):
    e = hr.shape[0]
    d = hr.shape[1]
    hid = We1.shape[1]
    w1a, w1b = We1[:d], We1[d:]
    grid = (e // block,)
    return pl.pallas_call(
        _edge_mlp_body,
        grid=grid,
        in_specs=[
            pl.BlockSpec((block, d), lambda i: (i, 0)),
            pl.BlockSpec((block, d), lambda i: (i, 0)),
            pl.BlockSpec((d, hid), lambda i: (0, 0)),
            pl.BlockSpec((d, hid), lambda i: (0, 0)),
            pl.BlockSpec((1, hid), lambda i: (0, 0)),
            pl.BlockSpec((hid, d), lambda i: (0, 0)),
            pl.BlockSpec((1, d), lambda i: (0, 0)),
        ],
        out_specs=pl.BlockSpec((block, d), lambda i: (i, 0)),
        out_shape=jax.ShapeDtypeStruct((e, d), jnp.float32),
    )(hr, hc, w1a, w1b, be1.reshape(1, -1), We2, be2.reshape(1, -1))


def _nb_mlp_body(vtv, nbfn, inv, ws1, wc1, wn1, b1, ws2, wc2, wn2, b2, m1, m2):
    ang = vtv[...] * inv[...]
    s = jnp.sin(ang)
    c = jnp.cos(ang)
    nb = nbfn[...]
    z1 = (jnp.dot(s, ws1[...], preferred_element_type=jnp.float32)
          + jnp.dot(c, wc1[...], preferred_element_type=jnp.float32)
          + jnp.dot(nb, wn1[...], preferred_element_type=jnp.float32))
    m1[...] = _silu(z1 + b1[...])
    z2 = (jnp.dot(s, ws2[...], preferred_element_type=jnp.float32)
          + jnp.dot(c, wc2[...], preferred_element_type=jnp.float32)
          + jnp.dot(nb, wn2[...], preferred_element_type=jnp.float32))
    m2[...] = _silu(z2 + b2[...])


def _nb_mlp(vtv, nbfn, Wp1, bp1, Wp2, bp2, block):
    e = vtv.shape[0]
    d = nbfn.shape[1]
    nf = VTV_NF
    dividers = jnp.arange(nf // 2, dtype=jnp.float32)
    div_term = jnp.exp(jnp.log(jnp.float32(10000.0)) * (2.0 * dividers / nf))
    inv = ((nf / 2.0) / div_term).reshape(1, nf // 2)
    ws1, wc1, wn1 = Wp1[0:nf:2], Wp1[1:nf:2], Wp1[nf:]
    ws2, wc2, wn2 = Wp2[0:nf:2], Wp2[1:nf:2], Wp2[nf:]
    grid = (e // block,)
    half = nf // 2
    return pl.pallas_call(
        _nb_mlp_body,
        grid=grid,
        in_specs=[
            pl.BlockSpec((block, 1), lambda i: (i, 0)),
            pl.BlockSpec((block, d), lambda i: (i, 0)),
            pl.BlockSpec((1, half), lambda i: (0, 0)),
            pl.BlockSpec((half, d), lambda i: (0, 0)),
            pl.BlockSpec((half, d), lambda i: (0, 0)),
            pl.BlockSpec((d, d), lambda i: (0, 0)),
            pl.BlockSpec((1, d), lambda i: (0, 0)),
            pl.BlockSpec((half, d), lambda i: (0, 0)),
            pl.BlockSpec((half, d), lambda i: (0, 0)),
            pl.BlockSpec((d, d), lambda i: (0, 0)),
            pl.BlockSpec((1, d), lambda i: (0, 0)),
        ],
        out_specs=[
            pl.BlockSpec((block, d), lambda i: (i, 0)),
            pl.BlockSpec((block, d), lambda i: (i, 0)),
        ],
        out_shape=[
            jax.ShapeDtypeStruct((e, d), jnp.float32),
            jax.ShapeDtypeStruct((e, d), jnp.float32),
        ],
    )(vtv.reshape(e, 1), nbfn, inv, ws1, wc1, wn1, bp1.reshape(1, -1),
      ws2, wc2, wn2, bp2.reshape(1, -1))


def _colsum_body(x, out):
    @pl.when(pl.program_id(0) == 0)
    def _():
        out[...] = jnp.zeros_like(out)
    out[...] += jnp.sum(x[...], axis=0, keepdims=True)


def _colsum(x, block):
    e, d = x.shape
    return pl.pallas_call(
        _colsum_body,
        grid=(e // block,),
        in_specs=[pl.BlockSpec((block, d), lambda i: (i, 0))],
        out_specs=pl.BlockSpec((1, d), lambda i: (0, 0)),
        out_shape=jax.ShapeDtypeStruct((1, d), jnp.float32),
    )(x)


def _t1w_body(ppgn, meanb, wa, wc1, bc1, wc2, t1, w):
    t = jnp.dot(ppgn[...], wa[...], preferred_element_type=jnp.float32) + meanb[...]
    t1[...] = t
    z = _silu(jnp.dot(t, wc1[...], preferred_element_type=jnp.float32) + bc1[...])
    w[...] = jnp.dot(z, wc2[...], preferred_element_type=jnp.float32)


def _t1w(ppgn, mean_row, Wa, Wb, bi, Wc1, bc1, Wc2, block):
    e, d = ppgn.shape
    hid = Wc1.shape[1]
    meanb = mean_row @ Wb + bi.reshape(1, -1)
    return pl.pallas_call(
        _t1w_body,
        grid=(e // block,),
        in_specs=[
            pl.BlockSpec((block, d), lambda i: (i, 0)),
            pl.BlockSpec((1, d), lambda i: (0, 0)),
            pl.BlockSpec((d, d), lambda i: (0, 0)),
            pl.BlockSpec((d, hid), lambda i: (0, 0)),
            pl.BlockSpec((1, hid), lambda i: (0, 0)),
            pl.BlockSpec((hid, 1), lambda i: (0, 0)),
        ],
        out_specs=[
            pl.BlockSpec((block, d), lambda i: (i, 0)),
            pl.BlockSpec((block, 1), lambda i: (i, 0)),
        ],
        out_shape=[
            jax.ShapeDtypeStruct((e, d), jnp.float32),
            jax.ShapeDtypeStruct((e, 1), jnp.float32),
        ],
    )(ppgn, meanb, Wa, Wc1, bc1.reshape(1, -1), Wc2)


def _node_mlp_body(t0, h, w1, b1, w2, b2, out):
    z = _silu(jnp.dot(t0[...], w1[...], preferred_element_type=jnp.float32) + b1[...])
    out[...] = h[...] + jnp.dot(z, w2[...], preferred_element_type=jnp.float32) + b2[...]


def _node_mlp(t0, h, Wn1, bn1, Wn2, bn2, block):
    n, d = t0.shape
    hid = Wn1.shape[1]
    return pl.pallas_call(
        _node_mlp_body,
        grid=(n // block,),
        in_specs=[
            pl.BlockSpec((block, d), lambda i: (i, 0)),
            pl.BlockSpec((block, d), lambda i: (i, 0)),
            pl.BlockSpec((d, hid), lambda i: (0, 0)),
            pl.BlockSpec((1, hid), lambda i: (0, 0)),
            pl.BlockSpec((hid, d), lambda i: (0, 0)),
            pl.BlockSpec((1, d), lambda i: (0, 0)),
        ],
        out_specs=pl.BlockSpec((block, d), lambda i: (i, 0)),
        out_shape=jax.ShapeDtypeStruct((n, d), jnp.float32),
    )(t0, h, Wn1, bn1.reshape(1, -1), Wn2, bn2.reshape(1, -1))


def kernel(h, x, edges, nb_edge, edge_attr, nb_num_nodes, We1, be1, We2, be2,
           Wp1, bp1, Wp2, bp2, Wa, Wb, bi, Wc1, bc1, Wc2, Wn1, bn1, Wn2, bn2):
    rows, cols = edges[0], edges[1]
    nbr, nbc = nb_edge[0], nb_edge[1]
    n = h.shape[0]
    e = edges.shape[1]
    enb = nb_edge.shape[1]

    cd = x[rows] - x[cols]
    vtv = jnp.sum(cd[nbr] * cd[nbc], axis=-1)

    efn = _edge_mlp(h[rows], h[cols], We1, be1, We2, be2, block=3200)
    nbfn = efn[nbr] * efn[nbc]

    m1, m2 = _nb_mlp(vtv, nbfn, Wp1, bp1, Wp2, bp2, block=3200)

    agg2 = jax.ops.segment_sum(m2, nbc, num_segments=e)
    ppgn = jax.ops.segment_sum(m1 * agg2[nbr], nbr, num_segments=e)

    mean_row = _colsum(ppgn, block=3200) / jnp.float32(e)
    t1, w = _t1w(ppgn, mean_row, Wa, Wb, bi, Wc1, bc1, Wc2, block=3200)

    trans = cd * w
    agg = jax.ops.segment_sum(trans, rows, num_segments=n)
    cnt = jax.ops.segment_sum(jnp.ones((e, 1), jnp.float32), rows, num_segments=n)
    x_out = x + agg / jnp.maximum(cnt, 1.0)

    t0 = jax.ops.segment_sum(t1, rows, num_segments=n)
    h_out = _node_mlp(t0, h, Wn1, bn1, Wn2, bn2, block=2000)
    return (h_out, x_out, edge_attr)


# speedup vs baseline: 1.0215x; 1.0215x over previous
import functools
import math

import jax
import jax.numpy as jnp
from jax.experimental import pallas as pl
from jax.experimental.pallas import tpu as pltpu

VTV_NF = 64


def _silu(z):
    return z * jax.nn.sigmoid(z)


def _edge_mlp_body(hr, hc, w1a, w1b, b1, w2, b2, out):
    z = jnp.dot(hr[...], w1a[...], preferred_element_type=jnp.float32)
    z += jnp.dot(hc[...], w1b[...], preferred_element_type=jnp.float32)
    z = _silu(z + b1[...])
    out[...] = jnp.dot(z, w2[...], preferred_element_type=jnp.float32) + b2[...]


def _edge_mlp(hr, hc, We1, be1, We2, be2, block):
    e = hr.shape[0]
    d = hr.shape[1]
    hid = We1.shape[1]
    w1a, w1b = We1[:d], We1[d:]
    grid = (e // block,)
    return pl.pallas_call(
        _edge_mlp_body,
        grid=grid,
        in_specs=[
            pl.BlockSpec((block, d), lambda i: (i, 0)),
            pl.BlockSpec((block, d), lambda i: (i, 0)),
            pl.BlockSpec((d, hid), lambda i: (0, 0)),
            pl.BlockSpec((d, hid), lambda i: (0, 0)),
            pl.BlockSpec((1, hid), lambda i: (0, 0)),
            pl.BlockSpec((hid, d), lambda i: (0, 0)),
            pl.BlockSpec((1, d), lambda i: (0, 0)),
        ],
        out_specs=pl.BlockSpec((block, d), lambda i: (i, 0)),
        out_shape=jax.ShapeDtypeStruct((e, d), jnp.float32),
    )(hr, hc, w1a, w1b, be1.reshape(1, -1), We2, be2.reshape(1, -1))


def _nb_mlp_body(vtv, nbfn, inv, ws1, wc1, wn1, b1, ws2, wc2, wn2, b2, m1, m2):
    ang = vtv[...] * inv[...]
    s = jnp.sin(ang)
    c = jnp.cos(ang)
    nb = nbfn[...]
    z1 = (jnp.dot(s, ws1[...], preferred_element_type=jnp.float32)
          + jnp.dot(c, wc1[...], preferred_element_type=jnp.float32)
          + jnp.dot(nb, wn1[...], preferred_element_type=jnp.float32))
    m1[...] = _silu(z1 + b1[...])
    z2 = (jnp.dot(s, ws2[...], preferred_element_type=jnp.float32)
          + jnp.dot(c, wc2[...], preferred_element_type=jnp.float32)
          + jnp.dot(nb, wn2[...], preferred_element_type=jnp.float32))
    m2[...] = _silu(z2 + b2[...])


def _nb_mlp(vtv, nbfn, Wp1, bp1, Wp2, bp2, block):
    e = vtv.shape[0]
    d = nbfn.shape[1]
    nf = VTV_NF
    dividers = jnp.arange(nf // 2, dtype=jnp.float32)
    div_term = jnp.exp(jnp.log(jnp.float32(10000.0)) * (2.0 * dividers / nf))
    inv = ((nf / 2.0) / div_term).reshape(1, nf // 2)
    ws1, wc1, wn1 = Wp1[0:nf:2], Wp1[1:nf:2], Wp1[nf:]
    ws2, wc2, wn2 = Wp2[0:nf:2], Wp2[1:nf:2], Wp2[nf:]
    grid = (e // block,)
    half = nf // 2
    return pl.pallas_call(
        _nb_mlp_body,
        grid=grid,
        in_specs=[
            pl.BlockSpec((block, 1), lambda i: (i, 0)),
            pl.BlockSpec((block, d), lambda i: (i, 0)),
            pl.BlockSpec((1, half), lambda i: (0, 0)),
            pl.BlockSpec((half, d), lambda i: (0, 0)),
            pl.BlockSpec((half, d), lambda i: (0, 0)),
            pl.BlockSpec((d, d), lambda i: (0, 0)),
            pl.BlockSpec((1, d), lambda i: (0, 0)),
            pl.BlockSpec((half, d), lambda i: (0, 0)),
            pl.BlockSpec((half, d), lambda i: (0, 0)),
            pl.BlockSpec((d, d), lambda i: (0, 0)),
            pl.BlockSpec((1, d), lambda i: (0, 0)),
        ],
        out_specs=[
            pl.BlockSpec((block, d), lambda i: (i, 0)),
            pl.BlockSpec((block, d), lambda i: (i, 0)),
        ],
        out_shape=[
            jax.ShapeDtypeStruct((e, d), jnp.float32),
            jax.ShapeDtypeStruct((e, d), jnp.float32),
        ],
    )(vtv.reshape(e, 1), nbfn, inv, ws1, wc1, wn1, bp1.reshape(1, -1),
      ws2, wc2, wn2, bp2.reshape(1, -1))


def _colsum_body(x, out):
    @pl.when(pl.program_id(0) == 0)
    def _():
        out[...] = jnp.zeros_like(out)
    out[...] += jnp.sum(x[...], axis=0, keepdims=True)


def _colsum(x, block):
    e, d = x.shape
    return pl.pallas_call(
        _colsum_body,
        grid=(e // block,),
        in_specs=[pl.BlockSpec((block, d), lambda i: (i, 0))],
        out_specs=pl.BlockSpec((1, d), lambda i: (0, 0)),
        out_shape=jax.ShapeDtypeStruct((1, d), jnp.float32),
    )(x)


def _t1w_body(ppgn, meanb, wa, wc1, bc1, wc2, t1, w):
    t = jnp.dot(ppgn[...], wa[...], preferred_element_type=jnp.float32) + meanb[...]
    t1[...] = t
    z = _silu(jnp.dot(t, wc1[...], preferred_element_type=jnp.float32) + bc1[...])
    w[...] = jnp.dot(z, wc2[...], preferred_element_type=jnp.float32)


def _t1w(ppgn, mean_row, Wa, Wb, bi, Wc1, bc1, Wc2, block):
    e, d = ppgn.shape
    hid = Wc1.shape[1]
    meanb = mean_row @ Wb + bi.reshape(1, -1)
    return pl.pallas_call(
        _t1w_body,
        grid=(e // block,),
        in_specs=[
            pl.BlockSpec((block, d), lambda i: (i, 0)),
            pl.BlockSpec((1, d), lambda i: (0, 0)),
            pl.BlockSpec((d, d), lambda i: (0, 0)),
            pl.BlockSpec((d, hid), lambda i: (0, 0)),
            pl.BlockSpec((1, hid), lambda i: (0, 0)),
            pl.BlockSpec((hid, 1), lambda i: (0, 0)),
        ],
        out_specs=[
            pl.BlockSpec((block, d), lambda i: (i, 0)),
            pl.BlockSpec((block, 1), lambda i: (i, 0)),
        ],
        out_shape=[
            jax.ShapeDtypeStruct((e, d), jnp.float32),
            jax.ShapeDtypeStruct((e, 1), jnp.float32),
        ],
    )(ppgn, meanb, Wa, Wc1, bc1.reshape(1, -1), Wc2)


def _node_mlp_body(t0, h, w1, b1, w2, b2, out):
    z = _silu(jnp.dot(t0[...], w1[...], preferred_element_type=jnp.float32) + b1[...])
    out[...] = h[...] + jnp.dot(z, w2[...], preferred_element_type=jnp.float32) + b2[...]


def _node_mlp(t0, h, Wn1, bn1, Wn2, bn2, block):
    n, d = t0.shape
    hid = Wn1.shape[1]
    return pl.pallas_call(
        _node_mlp_body,
        grid=(n // block,),
        in_specs=[
            pl.BlockSpec((block, d), lambda i: (i, 0)),
            pl.BlockSpec((block, d), lambda i: (i, 0)),
            pl.BlockSpec((d, hid), lambda i: (0, 0)),
            pl.BlockSpec((1, hid), lambda i: (0, 0)),
            pl.BlockSpec((hid, d), lambda i: (0, 0)),
            pl.BlockSpec((1, d), lambda i: (0, 0)),
        ],
        out_specs=pl.BlockSpec((block, d), lambda i: (i, 0)),
        out_shape=jax.ShapeDtypeStruct((n, d), jnp.float32),
    )(t0, h, Wn1, bn1.reshape(1, -1), Wn2, bn2.reshape(1, -1))


def kernel(h, x, edges, nb_edge, edge_attr, nb_num_nodes, We1, be1, We2, be2,
           Wp1, bp1, Wp2, bp2, Wa, Wb, bi, Wc1, bc1, Wc2, Wn1, bn1, Wn2, bn2):
    rows, cols = edges[0], edges[1]
    nbr, nbc = nb_edge[0], nb_edge[1]
    n = h.shape[0]
    e = edges.shape[1]
    enb = nb_edge.shape[1]

    cd = x[rows] - x[cols]
    vtv = jnp.sum(cd[nbr] * cd[nbc], axis=-1)

    efn = _edge_mlp(h[rows], h[cols], We1, be1, We2, be2, block=3200)
    nbfn = efn[nbr] * efn[nbc]

    m1, m2 = _nb_mlp(vtv, nbfn, Wp1, bp1, Wp2, bp2, block=3200)

    agg2 = jax.ops.segment_sum(m2, nbc, num_segments=e)
    ppgn = jax.ops.segment_sum(m1 * agg2[nbr], nbr, num_segments=e)

    mean_row = _colsum(ppgn, block=3200) / jnp.float32(e)
    t1, w = _t1w(ppgn, mean_row, Wa, Wb, bi, Wc1, bc1, Wc2, block=3200)

    packed = jnp.concatenate(
        [t1, cd * w, jnp.ones((e, 1), jnp.float32)], axis=1)
    psum = jax.ops.segment_sum(packed, rows, num_segments=n)
    t0, agg, cnt = psum[:, :128], psum[:, 128:131], psum[:, 131:132]
    x_out = x + agg / jnp.maximum(cnt, 1.0)
    h_out = _node_mlp(t0, h, Wn1, bn1, Wn2, bn2, block=2000)
    return (h_out, x_out, edge_attr)


# speedup vs baseline: 1.0716x; 1.0490x over previous
import functools
import math

import jax
import jax.numpy as jnp
from jax import lax
from jax.experimental import pallas as pl
from jax.experimental.pallas import tpu as pltpu
from jax.experimental.pallas import tpu_sc as plsc

VTV_NF = 64


def _silu(z):
    return z * jax.nn.sigmoid(z)


def _make_sc_rows_scatter(e, n_seg):
    accr = -(-(n_seg + 128) // 128) * 128
    epc = e // 2
    ept = epc // 16
    gb = 128
    ngb = -(-ept // gb)
    zr = 64
    rpt = accr // 16
    mesh = plsc.VectorSubcoreMesh(core_axis_name="c", subcore_axis_name="s")

    def body(t1, nrw, ridx, zrs, out, idx_v, rows_v, zbuf, acc, sem):
        t = lax.axis_index("s")
        ci = lax.axis_index("c")
        base = ci * epc + t * ept
        pltpu.sync_copy(ridx.at[pl.ds(base, ept)], idx_v.at[pl.ds(0, ept)])
        pltpu.sync_copy(zrs, zbuf)

        def zero_acc():
            for z in range(rpt // zr):
                pltpu.sync_copy(zbuf, acc.at[pl.ds(t * rpt + z * zr, zr)])

        def seg_ids(g, k):
            ev = idx_v[pl.ds(g * gb + k * 16, 16)]
            li = lax.iota(jnp.int32, 16) + (g * gb + k * 16)
            dumpv = jnp.full((16,), n_seg, jnp.int32) + t * 8
            return jnp.where(li < ept, ev, dumpv)

        def phase(src, out_slot):
            zero_acc()
            plsc.subcore_barrier()

            def gbatch(g, _):
                pltpu.async_copy(src.at[pl.ds(base + g * gb, gb)], rows_v,
                                 sem).wait()
                for k in range(gb // 16):
                    pltpu.sync_copy(rows_v.at[pl.ds(k * 16, 16)],
                                    acc.at[seg_ids(g, k)], add=True)
                return 0
            lax.fori_loop(0, ngb, gbatch, 0)
            plsc.subcore_barrier()
            pltpu.sync_copy(acc.at[pl.ds(t * rpt, rpt)],
                            out.at[out_slot, pl.ds(t * rpt, rpt)])
            plsc.subcore_barrier()

        phase(t1, ci)
        phase(nrw, 2 + ci)

    f = pl.kernel(
        body,
        out_type=jax.ShapeDtypeStruct((4, accr, 128), jnp.float32),
        mesh=mesh,
        scratch_types=[
            pltpu.VMEM((ngb * gb,), jnp.int32),
            pltpu.VMEM((gb, 128), jnp.float32),
            pltpu.VMEM((zr, 128), jnp.float32),
            pltpu.VMEM_SHARED((accr, 128), jnp.float32),
            pltpu.SemaphoreType.DMA,
        ],
    )
    zrs = jnp.zeros((zr, 128), jnp.float32)
    return lambda t1, nrw, ridx: f(t1, nrw, ridx, zrs)


def _edge_mlp_body(hr, hc, w1a, w1b, b1, w2, b2, out):
    z = jnp.dot(hr[...], w1a[...], preferred_element_type=jnp.float32)
    z += jnp.dot(hc[...], w1b[...], preferred_element_type=jnp.float32)
    z = _silu(z + b1[...])
    out[...] = jnp.dot(z, w2[...], preferred_element_type=jnp.float32) + b2[...]


def _edge_mlp(hr, hc, We1, be1, We2, be2, block):
    e = hr.shape[0]
    d = hr.shape[1]
    hid = We1.shape[1]
    w1a, w1b = We1[:d], We1[d:]
    grid = (e // block,)
    return pl.pallas_call(
        _edge_mlp_body,
        grid=grid,
        in_specs=[
            pl.BlockSpec((block, d), lambda i: (i, 0)),
            pl.BlockSpec((block, d), lambda i: (i, 0)),
            pl.BlockSpec((d, hid), lambda i: (0, 0)),
            pl.BlockSpec((d, hid), lambda i: (0, 0)),
            pl.BlockSpec((1, hid), lambda i: (0, 0)),
            pl.BlockSpec((hid, d), lambda i: (0, 0)),
            pl.BlockSpec((1, d), lambda i: (0, 0)),
        ],
        out_specs=pl.BlockSpec((block, d), lambda i: (i, 0)),
        out_shape=jax.ShapeDtypeStruct((e, d), jnp.float32),
    )(hr, hc, w1a, w1b, be1.reshape(1, -1), We2, be2.reshape(1, -1))


def _nb_mlp_body(vtv, nbfn, inv, ws1, wc1, wn1, b1, ws2, wc2, wn2, b2, m1, m2):
    ang = vtv[...] * inv[...]
    s = jnp.sin(ang)
    c = jnp.cos(ang)
    nb = nbfn[...]
    z1 = (jnp.dot(s, ws1[...], preferred_element_type=jnp.float32)
          + jnp.dot(c, wc1[...], preferred_element_type=jnp.float32)
          + jnp.dot(nb, wn1[...], preferred_element_type=jnp.float32))
    m1[...] = _silu(z1 + b1[...])
    z2 = (jnp.dot(s, ws2[...], preferred_element_type=jnp.float32)
          + jnp.dot(c, wc2[...], preferred_element_type=jnp.float32)
          + jnp.dot(nb, wn2[...], preferred_element_type=jnp.float32))
    m2[...] = _silu(z2 + b2[...])


def _nb_mlp(vtv, nbfn, Wp1, bp1, Wp2, bp2, block):
    e = vtv.shape[0]
    d = nbfn.shape[1]
    nf = VTV_NF
    dividers = jnp.arange(nf // 2, dtype=jnp.float32)
    div_term = jnp.exp(jnp.log(jnp.float32(10000.0)) * (2.0 * dividers / nf))
    inv = ((nf / 2.0) / div_term).reshape(1, nf // 2)
    ws1, wc1, wn1 = Wp1[0:nf:2], Wp1[1:nf:2], Wp1[nf:]
    ws2, wc2, wn2 = Wp2[0:nf:2], Wp2[1:nf:2], Wp2[nf:]
    grid = (e // block,)
    half = nf // 2
    return pl.pallas_call(
        _nb_mlp_body,
        grid=grid,
        in_specs=[
            pl.BlockSpec((block, 1), lambda i: (i, 0)),
            pl.BlockSpec((block, d), lambda i: (i, 0)),
            pl.BlockSpec((1, half), lambda i: (0, 0)),
            pl.BlockSpec((half, d), lambda i: (0, 0)),
            pl.BlockSpec((half, d), lambda i: (0, 0)),
            pl.BlockSpec((d, d), lambda i: (0, 0)),
            pl.BlockSpec((1, d), lambda i: (0, 0)),
            pl.BlockSpec((half, d), lambda i: (0, 0)),
            pl.BlockSpec((half, d), lambda i: (0, 0)),
            pl.BlockSpec((d, d), lambda i: (0, 0)),
            pl.BlockSpec((1, d), lambda i: (0, 0)),
        ],
        out_specs=[
            pl.BlockSpec((block, d), lambda i: (i, 0)),
            pl.BlockSpec((block, d), lambda i: (i, 0)),
        ],
        out_shape=[
            jax.ShapeDtypeStruct((e, d), jnp.float32),
            jax.ShapeDtypeStruct((e, d), jnp.float32),
        ],
    )(vtv.reshape(e, 1), nbfn, inv, ws1, wc1, wn1, bp1.reshape(1, -1),
      ws2, wc2, wn2, bp2.reshape(1, -1))


def _colsum_body(x, out):
    @pl.when(pl.program_id(0) == 0)
    def _():
        out[...] = jnp.zeros_like(out)
    out[...] += jnp.sum(x[...], axis=0, keepdims=True)


def _colsum(x, block):
    e, d = x.shape
    return pl.pallas_call(
        _colsum_body,
        grid=(e // block,),
        in_specs=[pl.BlockSpec((block, d), lambda i: (i, 0))],
        out_specs=pl.BlockSpec((1, d), lambda i: (0, 0)),
        out_shape=jax.ShapeDtypeStruct((1, d), jnp.float32),
    )(x)


def _t1w_body(ppgn, meanb, cd, wa, wc1, bc1, wc2, t1, nrw):
    t = jnp.dot(ppgn[...], wa[...], preferred_element_type=jnp.float32) + meanb[...]
    t1[...] = t
    z = _silu(jnp.dot(t, wc1[...], preferred_element_type=jnp.float32) + bc1[...])
    w = jnp.dot(z, wc2[...], preferred_element_type=jnp.float32)
    b = t.shape[0]
    nrw[...] = jnp.concatenate(
        [cd[...] * w, jnp.ones((b, 1), jnp.float32),
         jnp.zeros((b, 124), jnp.float32)], axis=1)


def _t1w(ppgn, mean_row, cd, Wa, Wb, bi, Wc1, bc1, Wc2, block, epad):
    e, d = ppgn.shape
    hid = Wc1.shape[1]
    meanb = mean_row @ Wb + bi.reshape(1, -1)
    return pl.pallas_call(
        _t1w_body,
        grid=(e // block,),
        in_specs=[
            pl.BlockSpec((block, d), lambda i: (i, 0)),
            pl.BlockSpec((1, d), lambda i: (0, 0)),
            pl.BlockSpec((block, 3), lambda i: (i, 0)),
            pl.BlockSpec((d, d), lambda i: (0, 0)),
            pl.BlockSpec((d, hid), lambda i: (0, 0)),
            pl.BlockSpec((1, hid), lambda i: (0, 0)),
            pl.BlockSpec((hid, 1), lambda i: (0, 0)),
        ],
        out_specs=[
            pl.BlockSpec((block, d), lambda i: (i, 0)),
            pl.BlockSpec((block, d), lambda i: (i, 0)),
        ],
        out_shape=[
            jax.ShapeDtypeStruct((epad, d), jnp.float32),
            jax.ShapeDtypeStruct((epad, d), jnp.float32),
        ],
    )(ppgn, meanb, cd, Wa, Wc1, bc1.reshape(1, -1), Wc2)


def _node_mlp_body(t0, h, w1, b1, w2, b2, out):
    z = _silu(jnp.dot(t0[...], w1[...], preferred_element_type=jnp.float32) + b1[...])
    out[...] = h[...] + jnp.dot(z, w2[...], preferred_element_type=jnp.float32) + b2[...]


def _node_mlp(t0, h, Wn1, bn1, Wn2, bn2, block):
    n, d = t0.shape
    hid = Wn1.shape[1]
    return pl.pallas_call(
        _node_mlp_body,
        grid=(n // block,),
        in_specs=[
            pl.BlockSpec((block, d), lambda i: (i, 0)),
            pl.BlockSpec((block, d), lambda i: (i, 0)),
            pl.BlockSpec((d, hid), lambda i: (0, 0)),
            pl.BlockSpec((1, hid), lambda i: (0, 0)),
            pl.BlockSpec((hid, d), lambda i: (0, 0)),
            pl.BlockSpec((1, d), lambda i: (0, 0)),
        ],
        out_specs=pl.BlockSpec((block, d), lambda i: (i, 0)),
        out_shape=jax.ShapeDtypeStruct((n, d), jnp.float32),
    )(t0, h, Wn1, bn1.reshape(1, -1), Wn2, bn2.reshape(1, -1))


def kernel(h, x, edges, nb_edge, edge_attr, nb_num_nodes, We1, be1, We2, be2,
           Wp1, bp1, Wp2, bp2, Wa, Wb, bi, Wc1, bc1, Wc2, Wn1, bn1, Wn2, bn2):
    rows, cols = edges[0], edges[1]
    nbr, nbc = nb_edge[0], nb_edge[1]
    n = h.shape[0]
    e = edges.shape[1]
    enb = nb_edge.shape[1]

    cd = x[rows] - x[cols]
    vtv = jnp.sum(cd[nbr] * cd[nbc], axis=-1)

    efn = _edge_mlp(h[rows], h[cols], We1, be1, We2, be2, block=3200)
    nbfn = efn[nbr] * efn[nbc]

    m1, m2 = _nb_mlp(vtv, nbfn, Wp1, bp1, Wp2, bp2, block=3200)

    agg2 = jax.ops.segment_sum(m2, nbc, num_segments=e)
    ppgn = jax.ops.segment_sum(m1 * agg2[nbr], nbr, num_segments=e)

    mean_row = _colsum(ppgn, block=3200) / jnp.float32(e)
    t1, nrw = _t1w(ppgn, mean_row, cd, Wa, Wb, bi, Wc1, bc1, Wc2,
                   block=2000, epad=e + 128)

    parts = _make_sc_rows_scatter(e, n)(t1, nrw, rows)
    t0 = parts[0, :n] + parts[1, :n]
    narrow = parts[2, :n, :4] + parts[3, :n, :4]
    x_out = x + narrow[:, :3] / jnp.maximum(narrow[:, 3:4], 1.0)
    h_out = _node_mlp(t0, h, Wn1, bn1, Wn2, bn2, block=2000)
    return (h_out, x_out, edge_attr)


# speedup vs baseline: 1.7632x; 1.6454x over previous
import functools
import math

import jax
import jax.numpy as jnp
from jax import lax
from jax.experimental import pallas as pl
from jax.experimental.pallas import tpu as pltpu
from jax.experimental.pallas import tpu_sc as plsc

VTV_NF = 64


def _silu(z):
    return z * jax.nn.sigmoid(z)


def _make_sc_rows_scatter(e, n_seg):
    accr = -(-(n_seg + 128) // 128) * 128
    epc = e // 2
    ept = epc // 16
    gb = 128
    ngb = -(-ept // gb)
    zr = 64
    rpt = accr // 16
    mesh = plsc.VectorSubcoreMesh(core_axis_name="c", subcore_axis_name="s")

    def body(t1, nrw, ridx, zrs, out, idx_v, rows_v, zbuf, acc, sem):
        t = lax.axis_index("s")
        ci = lax.axis_index("c")
        base = ci * epc + t * ept
        pltpu.sync_copy(ridx.at[pl.ds(base, ept)], idx_v.at[pl.ds(0, ept)])
        pltpu.sync_copy(zrs, zbuf)

        def zero_acc():
            for z in range(rpt // zr):
                pltpu.sync_copy(zbuf, acc.at[pl.ds(t * rpt + z * zr, zr)])

        def seg_ids(g, k):
            ev = idx_v[pl.ds(g * gb + k * 16, 16)]
            li = lax.iota(jnp.int32, 16) + (g * gb + k * 16)
            dumpv = jnp.full((16,), n_seg, jnp.int32) + t * 8
            return jnp.where(li < ept, ev, dumpv)

        def phase(src, out_slot):
            zero_acc()
            plsc.subcore_barrier()

            def gbatch(g, _):
                pltpu.async_copy(src.at[pl.ds(base + g * gb, gb)], rows_v,
                                 sem).wait()
                for k in range(gb // 16):
                    pltpu.sync_copy(rows_v.at[pl.ds(k * 16, 16)],
                                    acc.at[seg_ids(g, k)], add=True)
                return 0
            lax.fori_loop(0, ngb, gbatch, 0)
            plsc.subcore_barrier()
            pltpu.sync_copy(acc.at[pl.ds(t * rpt, rpt)],
                            out.at[out_slot, pl.ds(t * rpt, rpt)])
            plsc.subcore_barrier()

        phase(t1, ci)
        phase(nrw, 2 + ci)

    f = pl.kernel(
        body,
        out_type=jax.ShapeDtypeStruct((4, accr, 128), jnp.float32),
        mesh=mesh,
        scratch_types=[
            pltpu.VMEM((ngb * gb,), jnp.int32),
            pltpu.VMEM((gb, 128), jnp.float32),
            pltpu.VMEM((zr, 128), jnp.float32),
            pltpu.VMEM_SHARED((accr, 128), jnp.float32),
            pltpu.SemaphoreType.DMA,
        ],
    )
    zrs = jnp.zeros((zr, 128), jnp.float32)
    return lambda t1, nrw, ridx: f(t1, nrw, ridx, zrs)


def _make_sc_pair_gather(e, d):
    ept = e // 32
    gb = 200
    nb = ept // gb
    mesh = plsc.VectorSubcoreMesh(core_axis_name="c", subcore_axis_name="s")

    def body(hsrc, ridx, cidx, hr, hc, ir_v, ic_v, rows_v, sem):
        t = lax.axis_index("s")
        ci = lax.axis_index("c")
        base = (ci * 16 + t) * ept
        pltpu.sync_copy(ridx.at[pl.ds(base, ept)], ir_v)
        pltpu.sync_copy(cidx.at[pl.ds(base, ept)], ic_v)

        def bat(b, _):
            pltpu.async_copy(hsrc.at[ir_v.at[pl.ds(b * gb, gb)]], rows_v,
                             sem).wait()
            pltpu.sync_copy(rows_v, hr.at[pl.ds(base + b * gb, gb)])
            pltpu.async_copy(hsrc.at[ic_v.at[pl.ds(b * gb, gb)]], rows_v,
                             sem).wait()
            pltpu.sync_copy(rows_v, hc.at[pl.ds(base + b * gb, gb)])
            return 0
        lax.fori_loop(0, nb, bat, 0)

    return pl.kernel(
        body,
        out_type=[jax.ShapeDtypeStruct((e, d), jnp.float32),
                  jax.ShapeDtypeStruct((e, d), jnp.float32)],
        mesh=mesh,
        scratch_types=[
            pltpu.VMEM((ept,), jnp.int32),
            pltpu.VMEM((ept,), jnp.int32),
            pltpu.VMEM((gb, d), jnp.float32),
            pltpu.SemaphoreType.DMA,
        ],
    )


def _make_sc_vtv(enb, e):
    ept = enb // 32
    mesh = plsc.VectorSubcoreMesh(core_axis_name="c", subcore_axis_name="s")

    def body(cdx, cdy, cdz, ridx, cidx, vtv, ir_v, ic_v,
             ax, ay, az, bx, by, bz, out_v, sem):
        t = lax.axis_index("s")
        ci = lax.axis_index("c")
        base = (ci * 16 + t) * ept
        pltpu.sync_copy(ridx.at[pl.ds(base, ept)], ir_v)
        pltpu.sync_copy(cidx.at[pl.ds(base, ept)], ic_v)
        for src, idx, dst in ((cdx, ir_v, ax), (cdy, ir_v, ay),
                              (cdz, ir_v, az), (cdx, ic_v, bx),
                              (cdy, ic_v, by), (cdz, ic_v, bz)):
            pltpu.async_copy(src.at[idx], dst, sem).wait()

        def dot(i, _):
            s = pl.ds(i * 16, 16)
            out_v[s] = ax[s] * bx[s] + ay[s] * by[s] + az[s] * bz[s]
            return 0
        lax.fori_loop(0, ept // 16, dot, 0)
        pltpu.sync_copy(out_v, vtv.at[pl.ds(base, ept)])

    return pl.kernel(
        body,
        out_type=jax.ShapeDtypeStruct((enb,), jnp.float32),
        mesh=mesh,
        scratch_types=[
            pltpu.VMEM((ept,), jnp.int32),
            pltpu.VMEM((ept,), jnp.int32),
        ] + [pltpu.VMEM((ept,), jnp.float32)] * 7 + [
            pltpu.SemaphoreType.DMA,
        ],
    )


def _edge_mlp_body(hr, hc, w1a, w1b, b1, w2, b2, out):
    z = jnp.dot(hr[...], w1a[...], preferred_element_type=jnp.float32)
    z += jnp.dot(hc[...], w1b[...], preferred_element_type=jnp.float32)
    z = _silu(z + b1[...])
    out[...] = jnp.dot(z, w2[...], preferred_element_type=jnp.float32) + b2[...]


def _edge_mlp(hr, hc, We1, be1, We2, be2, block):
    e = hr.shape[0]
    d = hr.shape[1]
    hid = We1.shape[1]
    w1a, w1b = We1[:d], We1[d:]
    grid = (e // block,)
    return pl.pallas_call(
        _edge_mlp_body,
        grid=grid,
        in_specs=[
            pl.BlockSpec((block, d), lambda i: (i, 0)),
            pl.BlockSpec((block, d), lambda i: (i, 0)),
            pl.BlockSpec((d, hid), lambda i: (0, 0)),
            pl.BlockSpec((d, hid), lambda i: (0, 0)),
            pl.BlockSpec((1, hid), lambda i: (0, 0)),
            pl.BlockSpec((hid, d), lambda i: (0, 0)),
            pl.BlockSpec((1, d), lambda i: (0, 0)),
        ],
        out_specs=pl.BlockSpec((block, d), lambda i: (i, 0)),
        out_shape=jax.ShapeDtypeStruct((e, d), jnp.float32),
    )(hr, hc, w1a, w1b, be1.reshape(1, -1), We2, be2.reshape(1, -1))


def _nb_mlp_body(vtv, nbfn, inv, ws1, wc1, wn1, b1, ws2, wc2, wn2, b2, m1, m2):
    ang = vtv[...] * inv[...]
    s = jnp.sin(ang)
    c = jnp.cos(ang)
    nb = nbfn[...]
    z1 = (jnp.dot(s, ws1[...], preferred_element_type=jnp.float32)
          + jnp.dot(c, wc1[...], preferred_element_type=jnp.float32)
          + jnp.dot(nb, wn1[...], preferred_element_type=jnp.float32))
    m1[...] = _silu(z1 + b1[...])
    z2 = (jnp.dot(s, ws2[...], preferred_element_type=jnp.float32)
          + jnp.dot(c, wc2[...], preferred_element_type=jnp.float32)
          + jnp.dot(nb, wn2[...], preferred_element_type=jnp.float32))
    m2[...] = _silu(z2 + b2[...])


def _nb_mlp(vtv, nbfn, Wp1, bp1, Wp2, bp2, block):
    e = vtv.shape[0]
    d = nbfn.shape[1]
    nf = VTV_NF
    dividers = jnp.arange(nf // 2, dtype=jnp.float32)
    div_term = jnp.exp(jnp.log(jnp.float32(10000.0)) * (2.0 * dividers / nf))
    inv = ((nf / 2.0) / div_term).reshape(1, nf // 2)
    ws1, wc1, wn1 = Wp1[0:nf:2], Wp1[1:nf:2], Wp1[nf:]
    ws2, wc2, wn2 = Wp2[0:nf:2], Wp2[1:nf:2], Wp2[nf:]
    grid = (e // block,)
    half = nf // 2
    return pl.pallas_call(
        _nb_mlp_body,
        grid=grid,
        in_specs=[
            pl.BlockSpec((block, 1), lambda i: (i, 0)),
            pl.BlockSpec((block, d), lambda i: (i, 0)),
            pl.BlockSpec((1, half), lambda i: (0, 0)),
            pl.BlockSpec((half, d), lambda i: (0, 0)),
            pl.BlockSpec((half, d), lambda i: (0, 0)),
            pl.BlockSpec((d, d), lambda i: (0, 0)),
            pl.BlockSpec((1, d), lambda i: (0, 0)),
            pl.BlockSpec((half, d), lambda i: (0, 0)),
            pl.BlockSpec((half, d), lambda i: (0, 0)),
            pl.BlockSpec((d, d), lambda i: (0, 0)),
            pl.BlockSpec((1, d), lambda i: (0, 0)),
        ],
        out_specs=[
            pl.BlockSpec((block, d), lambda i: (i, 0)),
            pl.BlockSpec((block, d), lambda i: (i, 0)),
        ],
        out_shape=[
            jax.ShapeDtypeStruct((e, d), jnp.float32),
            jax.ShapeDtypeStruct((e, d), jnp.float32),
        ],
    )(vtv.reshape(e, 1), nbfn, inv, ws1, wc1, wn1, bp1.reshape(1, -1),
      ws2, wc2, wn2, bp2.reshape(1, -1))


def _colsum_body(x, out):
    @pl.when(pl.program_id(0) == 0)
    def _():
        out[...] = jnp.zeros_like(out)
    out[...] += jnp.sum(x[...], axis=0, keepdims=True)


def _colsum(x, block):
    e, d = x.shape
    return pl.pallas_call(
        _colsum_body,
        grid=(e // block,),
        in_specs=[pl.BlockSpec((block, d), lambda i: (i, 0))],
        out_specs=pl.BlockSpec((1, d), lambda i: (0, 0)),
        out_shape=jax.ShapeDtypeStruct((1, d), jnp.float32),
    )(x)


def _t1w_body(ppgn, meanb, cd, wa, wc1, bc1, wc2, t1, nrw):
    t = jnp.dot(ppgn[...], wa[...], preferred_element_type=jnp.float32) + meanb[...]
    t1[...] = t
    z = _silu(jnp.dot(t, wc1[...], preferred_element_type=jnp.float32) + bc1[...])
    w = jnp.dot(z, wc2[...], preferred_element_type=jnp.float32)
    b = t.shape[0]
    nrw[...] = jnp.concatenate(
        [cd[...] * w, jnp.ones((b, 1), jnp.float32),
         jnp.zeros((b, 124), jnp.float32)], axis=1)


def _t1w(ppgn, mean_row, cd, Wa, Wb, bi, Wc1, bc1, Wc2, block, epad):
    e, d = ppgn.shape
    hid = Wc1.shape[1]
    meanb = mean_row @ Wb + bi.reshape(1, -1)
    return pl.pallas_call(
        _t1w_body,
        grid=(e // block,),
        in_specs=[
            pl.BlockSpec((block, d), lambda i: (i, 0)),
            pl.BlockSpec((1, d), lambda i: (0, 0)),
            pl.BlockSpec((block, 3), lambda i: (i, 0)),
            pl.BlockSpec((d, d), lambda i: (0, 0)),
            pl.BlockSpec((d, hid), lambda i: (0, 0)),
            pl.BlockSpec((1, hid), lambda i: (0, 0)),
            pl.BlockSpec((hid, 1), lambda i: (0, 0)),
        ],
        out_specs=[
            pl.BlockSpec((block, d), lambda i: (i, 0)),
            pl.BlockSpec((block, d), lambda i: (i, 0)),
        ],
        out_shape=[
            jax.ShapeDtypeStruct((epad, d), jnp.float32),
            jax.ShapeDtypeStruct((epad, d), jnp.float32),
        ],
    )(ppgn, meanb, cd, Wa, Wc1, bc1.reshape(1, -1), Wc2)


def _node_mlp_body(t0, h, w1, b1, w2, b2, out):
    z = _silu(jnp.dot(t0[...], w1[...], preferred_element_type=jnp.float32) + b1[...])
    out[...] = h[...] + jnp.dot(z, w2[...], preferred_element_type=jnp.float32) + b2[...]


def _node_mlp(t0, h, Wn1, bn1, Wn2, bn2, block):
    n, d = t0.shape
    hid = Wn1.shape[1]
    return pl.pallas_call(
        _node_mlp_body,
        grid=(n // block,),
        in_specs=[
            pl.BlockSpec((block, d), lambda i: (i, 0)),
            pl.BlockSpec((block, d), lambda i: (i, 0)),
            pl.BlockSpec((d, hid), lambda i: (0, 0)),
            pl.BlockSpec((1, hid), lambda i: (0, 0)),
            pl.BlockSpec((hid, d), lambda i: (0, 0)),
            pl.BlockSpec((1, d), lambda i: (0, 0)),
        ],
        out_specs=pl.BlockSpec((block, d), lambda i: (i, 0)),
        out_shape=jax.ShapeDtypeStruct((n, d), jnp.float32),
    )(t0, h, Wn1, bn1.reshape(1, -1), Wn2, bn2.reshape(1, -1))


def kernel(h, x, edges, nb_edge, edge_attr, nb_num_nodes, We1, be1, We2, be2,
           Wp1, bp1, Wp2, bp2, Wa, Wb, bi, Wc1, bc1, Wc2, Wn1, bn1, Wn2, bn2):
    rows, cols = edges[0], edges[1]
    nbr, nbc = nb_edge[0], nb_edge[1]
    n = h.shape[0]
    e = edges.shape[1]
    enb = nb_edge.shape[1]

    cd = x[rows] - x[cols]
    vtv = _make_sc_vtv(enb, e)(
        cd[:, 0], cd[:, 1], cd[:, 2], nbr, nbc)

    hr, hc = _make_sc_pair_gather(e, h.shape[1])(h, rows, cols)
    efn = _edge_mlp(hr, hc, We1, be1, We2, be2, block=3200)
    nbfn = efn[nbr] * efn[nbc]

    m1, m2 = _nb_mlp(vtv, nbfn, Wp1, bp1, Wp2, bp2, block=3200)

    agg2 = jax.ops.segment_sum(m2, nbc, num_segments=e)
    ppgn = jax.ops.segment_sum(m1 * agg2[nbr], nbr, num_segments=e)

    mean_row = _colsum(ppgn, block=3200) / jnp.float32(e)
    t1, nrw = _t1w(ppgn, mean_row, cd, Wa, Wb, bi, Wc1, bc1, Wc2,
                   block=2000, epad=e + 128)

    parts = _make_sc_rows_scatter(e, n)(t1, nrw, rows)
    t0 = parts[0, :n] + parts[1, :n]
    narrow = parts[2, :n, :4] + parts[3, :n, :4]
    x_out = x + narrow[:, :3] / jnp.maximum(narrow[:, 3:4], 1.0)
    h_out = _node_mlp(t0, h, Wn1, bn1, Wn2, bn2, block=2000)
    return (h_out, x_out, edge_attr)
